# baseline (speedup 1.0000x reference)
import functools

import jax
import jax.numpy as jnp
from jax import lax
from jax.experimental import pallas as pl
from jax.experimental.pallas import tpu as pltpu
from jax.experimental.pallas import tpu_sc as plsc


def _make_sc_add(N, S, D, num_cores, num_subcores):
    NW = num_cores * num_subcores
    rows_per_w = S // NW
    T = 8
    steps = rows_per_w // T
    VPR = D // 16

    mesh = plsc.VectorSubcoreMesh(core_axis_name="c", subcore_axis_name="s")

    @functools.partial(
        pl.kernel,
        out_type=jax.ShapeDtypeStruct((N, S, D), jnp.float32),
        mesh=mesh,
        scratch_types=[
            pltpu.VMEM((T, D), jnp.float32),
            pltpu.VMEM((T, D), jnp.float32),
        ]
        + [pltpu.VMEM((T, D), jnp.float32) for _ in range(N)]
        + [pltpu.SemaphoreType.DMA for _ in range(2 + 2 * N)],
    )
    def sc_add(x_hbm, t_hbm, o_hbm, tb0, tb1, *rest):
        xb = rest[:N]
        ts = rest[N:N + 2]
        xs = rest[N + 2:N + 2 + N]
        ss = rest[N + 2 + N:]

        wid = lax.axis_index("s") * num_cores + lax.axis_index("c")
        row0 = wid * rows_per_w

        def t_slice(si):
            return t_hbm.at[pl.ds(row0 + si * T, T), :]

        def x_slice(si, n):
            return x_hbm.at[n, pl.ds(row0 + si * T, T), :]

        def o_slice(si, n):
            return o_hbm.at[n, pl.ds(row0 + si * T, T), :]

        def add_chunk(xbuf, tbuf):
            @plsc.parallel_loop(0, T * VPR, unroll=8)
            def _add(i):
                r = i // VPR
                c = (i % VPR) * 16
                plsc.addupdate(xbuf.at[r, pl.ds(c, 16)], tbuf[r, pl.ds(c, 16)])

        def reload(si_next, m):
            pltpu.make_async_copy(xb[m], o_slice(si_next - 1, m), ss[m]).wait()
            pltpu.make_async_copy(x_slice(si_next, m), xb[m], xs[m]).start()

        def group(si, tb_this, ts_this, tb_other, ts_other):
            @pl.when(si + 1 < steps)
            def _():
                pltpu.make_async_copy(t_slice(si + 1), tb_other, ts_other).start()

            pltpu.make_async_copy(t_slice(si), tb_this, ts_this).wait()

            for n in range(N):
                pltpu.make_async_copy(x_slice(si, n), xb[n], xs[n]).wait()
                add_chunk(xb[n], tb_this)
                pltpu.make_async_copy(xb[n], o_slice(si, n), ss[n]).start()
                if n >= 2:
                    @pl.when(si + 1 < steps)
                    def _():
                        reload(si + 1, n - 2)
            for m in range(max(0, N - 2), N):
                @pl.when(si + 1 < steps)
                def _():
                    reload(si + 1, m)

        pltpu.make_async_copy(t_slice(0), tb0, ts[0]).start()
        for n in range(N):
            pltpu.make_async_copy(x_slice(0, n), xb[n], xs[n]).start()

        def body(so, c):
            group(2 * so, tb0, ts[0], tb1, ts[1])
            group(2 * so + 1, tb1, ts[1], tb0, ts[0])
            return c

        lax.fori_loop(0, steps // 2, body, 0)

        for n in range(N):
            pltpu.make_async_copy(xb[n], o_slice(steps - 1, n), ss[n]).wait()

    return sc_add


def kernel(x, pos_table):
    N, S, D = x.shape
    info = plsc.get_sparse_core_info()
    sc_add = _make_sc_add(N, S, D, info.num_cores, info.num_subcores)
    return sc_add(x, pos_table)


# speedup vs baseline: 1.0264x; 1.0264x over previous
import functools

import jax
import jax.numpy as jnp
from jax import lax
from jax.experimental import pallas as pl
from jax.experimental.pallas import tpu as pltpu
from jax.experimental.pallas import tpu_sc as plsc


def _make_sc_add(N, S, D, num_cores, num_subcores):
    NW = num_cores * num_subcores
    rows_per_w = S // NW
    T = 16
    steps = rows_per_w // T
    VPR = D // 16

    mesh = plsc.VectorSubcoreMesh(core_axis_name="c", subcore_axis_name="s")

    @functools.partial(
        pl.kernel,
        out_type=jax.ShapeDtypeStruct((N, S, D), jnp.float32),
        mesh=mesh,
        scratch_types=[
            pltpu.VMEM((T, D), jnp.float32),
            pltpu.VMEM((T, D), jnp.float32),
        ]
        + [pltpu.VMEM((T, D), jnp.float32) for _ in range(N)]
        + [pltpu.SemaphoreType.DMA for _ in range(2 + 2 * N)],
    )
    def sc_add(x_hbm, t_hbm, o_hbm, tb0, tb1, *rest):
        xb = rest[:N]
        ts = rest[N:N + 2]
        xs = rest[N + 2:N + 2 + N]
        ss = rest[N + 2 + N:]

        wid = lax.axis_index("s") * num_cores + lax.axis_index("c")
        row0 = wid * rows_per_w

        def t_slice(si):
            return t_hbm.at[pl.ds(row0 + si * T, T), :]

        def x_slice(si, n):
            return x_hbm.at[n, pl.ds(row0 + si * T, T), :]

        def o_slice(si, n):
            return o_hbm.at[n, pl.ds(row0 + si * T, T), :]

        def add_rows(xbuf, tbuf, r0, r1):
            @plsc.parallel_loop(r0 * VPR, r1 * VPR, unroll=8)
            def _add(i):
                r = i // VPR
                c = (i % VPR) * 16
                plsc.addupdate(xbuf.at[r, pl.ds(c, 16)], tbuf[r, pl.ds(c, 16)])

        def reload(si_next, m):
            pltpu.make_async_copy(xb[m], o_slice(si_next - 1, m), ss[m]).wait()
            pltpu.make_async_copy(x_slice(si_next, m), xb[m], xs[m]).start()

        def group(si, tb_this, ts_this, tb_other, ts_other):
            @pl.when(si + 1 < steps)
            def _():
                pltpu.make_async_copy(t_slice(si + 1), tb_other, ts_other).start()

            pltpu.make_async_copy(t_slice(si), tb_this, ts_this).wait()

            for n in range(N):
                pltpu.make_async_copy(x_slice(si, n), xb[n], xs[n]).wait()
                H = T // 2
                add_rows(xb[n], tb_this, 0, H)
                pltpu.make_async_copy(
                    xb[n].at[pl.ds(0, H), :],
                    o_hbm.at[n, pl.ds(row0 + si * T, H), :],
                    ss[n],
                ).start()
                add_rows(xb[n], tb_this, H, T)
                pltpu.make_async_copy(
                    xb[n].at[pl.ds(H, H), :],
                    o_hbm.at[n, pl.ds(row0 + si * T + H, H), :],
                    ss[n],
                ).start()
                if n >= 2:
                    @pl.when(si + 1 < steps)
                    def _():
                        reload(si + 1, n - 2)
            for m in range(max(0, N - 2), N):
                @pl.when(si + 1 < steps)
                def _():
                    reload(si + 1, m)

        pltpu.make_async_copy(t_slice(0), tb0, ts[0]).start()
        for n in range(N):
            pltpu.make_async_copy(x_slice(0, n), xb[n], xs[n]).start()

        def body(so, c):
            group(2 * so, tb0, ts[0], tb1, ts[1])
            group(2 * so + 1, tb1, ts[1], tb0, ts[0])
            return c

        lax.fori_loop(0, steps // 2, body, 0)

        for n in range(N):
            pltpu.make_async_copy(xb[n], o_slice(steps - 1, n), ss[n]).wait()

    return sc_add


def kernel(x, pos_table):
    N, S, D = x.shape
    info = plsc.get_sparse_core_info()
    sc_add = _make_sc_add(N, S, D, info.num_cores, info.num_subcores)
    return sc_add(x, pos_table)


# speedup vs baseline: 1.0494x; 1.0225x over previous
import functools

import jax
import jax.numpy as jnp
from jax import lax
from jax.experimental import pallas as pl
from jax.experimental.pallas import tpu as pltpu
from jax.experimental.pallas import tpu_sc as plsc


def _make_sc_add(N, S, D, num_cores, num_subcores):
    NW = num_cores * num_subcores
    rows_per_w = S // NW
    T = 16
    steps = rows_per_w // T
    VPR = D // 16

    mesh = plsc.VectorSubcoreMesh(core_axis_name="c", subcore_axis_name="s")

    @functools.partial(
        pl.kernel,
        out_type=jax.ShapeDtypeStruct((N, S, D), jnp.float32),
        mesh=mesh,
        scratch_types=[
            pltpu.VMEM((T, D), jnp.float32),
            pltpu.VMEM((T, D), jnp.float32),
        ]
        + [pltpu.VMEM((T, D), jnp.float32) for _ in range(N)]
        + [pltpu.SemaphoreType.DMA for _ in range(2 + 2 * N)],
    )
    def sc_add(x_hbm, t_hbm, o_hbm, tb0, tb1, *rest):
        xb = rest[:N]
        ts = rest[N:N + 2]
        xs = rest[N + 2:N + 2 + N]
        ss = rest[N + 2 + N:]

        wid = lax.axis_index("s") * num_cores + lax.axis_index("c")
        row0 = wid * rows_per_w

        def t_slice(si):
            return t_hbm.at[pl.ds(row0 + si * T, T), :]

        def x_slice(si, n):
            return x_hbm.at[n, pl.ds(row0 + si * T, T), :]

        def o_slice(si, n):
            return o_hbm.at[n, pl.ds(row0 + si * T, T), :]

        def add_chunk(xbuf, tbuf):
            @plsc.parallel_loop(0, T * VPR, unroll=8)
            def _add(i):
                r = i // VPR
                c = (i % VPR) * 16
                plsc.addupdate(xbuf.at[r, pl.ds(c, 16)], tbuf[r, pl.ds(c, 16)])

        def reload(si_next, m):
            pltpu.make_async_copy(xb[m], o_slice(si_next - 1, m), ss[m]).wait()
            pltpu.make_async_copy(x_slice(si_next, m), xb[m], xs[m]).start()

        def group(si, tb_this, ts_this, tb_other, ts_other):
            @pl.when(si + 1 < steps)
            def _():
                pltpu.make_async_copy(t_slice(si + 1), tb_other, ts_other).start()

            pltpu.make_async_copy(t_slice(si), tb_this, ts_this).wait()

            for n in range(N):
                pltpu.make_async_copy(x_slice(si, n), xb[n], xs[n]).wait()
                add_chunk(xb[n], tb_this)
                pltpu.make_async_copy(xb[n], o_slice(si, n), ss[n]).start()
                if n >= 2:
                    @pl.when(si + 1 < steps)
                    def _():
                        reload(si + 1, n - 2)
            for m in range(max(0, N - 2), N):
                @pl.when(si + 1 < steps)
                def _():
                    reload(si + 1, m)

        pltpu.make_async_copy(t_slice(0), tb0, ts[0]).start()
        for n in range(N):
            pltpu.make_async_copy(x_slice(0, n), xb[n], xs[n]).start()

        def body(so, c):
            group(2 * so, tb0, ts[0], tb1, ts[1])
            group(2 * so + 1, tb1, ts[1], tb0, ts[0])
            return c

        lax.fori_loop(0, steps // 2, body, 0)

        for n in range(N):
            pltpu.make_async_copy(xb[n], o_slice(steps - 1, n), ss[n]).wait()

    return sc_add


def kernel(x, pos_table):
    N, S, D = x.shape
    info = plsc.get_sparse_core_info()
    sc_add = _make_sc_add(N, S, D, info.num_cores, info.num_subcores)
    return sc_add(x, pos_table)


# speedup vs baseline: 1.1072x; 1.0550x over previous
import functools

import jax
import jax.numpy as jnp
from jax import lax
from jax.experimental import pallas as pl
from jax.experimental.pallas import tpu as pltpu
from jax.experimental.pallas import tpu_sc as plsc


def _make_sc_add(N, S, D, num_cores, num_subcores):
    NW = num_cores * num_subcores
    rows_per_w = S // NW
    T = 8
    steps = rows_per_w // T
    VPR = D // 16

    mesh = plsc.VectorSubcoreMesh(core_axis_name="c", subcore_axis_name="s")

    @functools.partial(
        pl.kernel,
        out_type=jax.ShapeDtypeStruct((N, S, D), jnp.float32),
        mesh=mesh,
        scratch_types=[
            pltpu.VMEM((T, D), jnp.float32),
            pltpu.VMEM((T, D), jnp.float32),
        ]
        + [pltpu.VMEM((T, D), jnp.float32) for _ in range(2 * N)]
        + [pltpu.SemaphoreType.DMA for _ in range(2 + 4 * N)],
    )
    def sc_add(x_hbm, t_hbm, o_hbm, tb0, tb1, *rest):
        xball = rest[:2 * N]
        ts = rest[2 * N:2 * N + 2]
        xsall = rest[2 * N + 2:2 * N + 2 + 2 * N]
        ssall = rest[2 * N + 2 + 2 * N:]

        setA = (xball[:N], xsall[:N], ssall[:N])
        setB = (xball[N:], xsall[N:], ssall[N:])

        wid = lax.axis_index("s") * num_cores + lax.axis_index("c")
        row0 = wid * rows_per_w

        def t_slice(si):
            return t_hbm.at[pl.ds(row0 + si * T, T), :]

        def x_slice(si, n):
            return x_hbm.at[n, pl.ds(row0 + si * T, T), :]

        def o_slice(si, n):
            return o_hbm.at[n, pl.ds(row0 + si * T, T), :]

        def add_chunk(xbuf, tbuf):
            @plsc.parallel_loop(0, T * VPR, unroll=8)
            def _add(i):
                r = i // VPR
                c = (i % VPR) * 16
                plsc.addupdate(xbuf.at[r, pl.ds(c, 16)], tbuf[r, pl.ds(c, 16)])

        def group(si, this, other, tb_this, ts_this, tb_other, ts_other):
            xb_t, xs_t, ss_t = this
            xb_o, xs_o, ss_o = other

            @pl.when(si + 1 < steps)
            def _():
                pltpu.make_async_copy(t_slice(si + 1), tb_other, ts_other).start()
                for n in range(N):
                    @pl.when(si >= 1)
                    def _():
                        pltpu.make_async_copy(
                            xb_o[n], o_slice(si - 1, n), ss_o[n]).wait()
                    pltpu.make_async_copy(
                        x_slice(si + 1, n), xb_o[n], xs_o[n]).start()

            pltpu.make_async_copy(t_slice(si), tb_this, ts_this).wait()
            for n in range(N):
                pltpu.make_async_copy(x_slice(si, n), xb_t[n], xs_t[n]).wait()
                add_chunk(xb_t[n], tb_this)
                pltpu.make_async_copy(xb_t[n], o_slice(si, n), ss_t[n]).start()

        pltpu.make_async_copy(t_slice(0), tb0, ts[0]).start()
        for n in range(N):
            pltpu.make_async_copy(x_slice(0, n), setA[0][n], setA[1][n]).start()

        def body(so, c):
            group(2 * so, setA, setB, tb0, ts[0], tb1, ts[1])
            group(2 * so + 1, setB, setA, tb1, ts[1], tb0, ts[0])
            return c

        lax.fori_loop(0, steps // 2, body, 0)

        for si in (steps - 2, steps - 1):
            st = setB if si % 2 else setA
            for n in range(N):
                pltpu.make_async_copy(st[0][n], o_slice(si, n), st[2][n]).wait()

    return sc_add


def kernel(x, pos_table):
    N, S, D = x.shape
    info = plsc.get_sparse_core_info()
    sc_add = _make_sc_add(N, S, D, info.num_cores, info.num_subcores)
    return sc_add(x, pos_table)
